# baseline (speedup 1.0000x reference)
import functools

import jax
import jax.numpy as jnp
from jax import lax
from jax.experimental import pallas as pl
from jax.experimental.pallas import tpu as pltpu
from jax.experimental.pallas import tpu_sc as plsc

N = 10000
E = 320000
IN_DIM = 128
H_DIM = 64
HID = 512

NC = 2
NS = 16
NW = NC * NS
C = 128
GB = 8
NG = 10
T = NG * GB * C
EPAD = NW * T
NPAD = 10240
RT = NPAD // NS
RB = 640
NBLK = NPAD // RB
DEGW = 16


def _make_agg(d: int, with_deg: bool):
    out_type = [jax.ShapeDtypeStruct((NC, NPAD, d), jnp.float32)]
    scratch = [
        pltpu.VMEM((GB, C), jnp.int32),
        pltpu.VMEM((GB, C), jnp.int32),
        pltpu.VMEM((C, d), jnp.float32),
        pltpu.VMEM((C, d), jnp.float32),
        pltpu.VMEM_SHARED((NPAD, d), jnp.float32),
        pltpu.SemaphoreType.DMA,
        pltpu.SemaphoreType.DMA,
        pltpu.SemaphoreType.DMA,
        pltpu.SemaphoreType.DMA,
        pltpu.SemaphoreType.DMA,
    ]
    if with_deg:
        out_type.append(jax.ShapeDtypeStruct((NC, NPAD, DEGW), jnp.float32))
        scratch += [
            pltpu.VMEM((C, DEGW), jnp.float32),
            pltpu.VMEM((16, DEGW), jnp.float32),
            pltpu.VMEM_SHARED((NPAD, DEGW), jnp.float32),
        ]

    mesh = plsc.VectorSubcoreMesh(core_axis_name="c", subcore_axis_name="s")
    params = pltpu.CompilerParams(use_tc_tiling_on_sc=False)

    @functools.partial(pl.kernel, out_type=tuple(out_type), mesh=mesh,
                       scratch_types=scratch, compiler_params=params)
    def agg(table_hbm, src_hbm, dst_hbm, zeros_hbm, *refs):
        if with_deg:
            (psum_hbm, pdeg_hbm, srcv, dstv, rows0, rows1, acc,
             semg0, semg1, sems0, sems1, semd, ones, zdb, dacc) = refs
        else:
            (psum_hbm, srcv, dstv, rows0, rows1, acc,
             semg0, semg1, sems0, sems1, semd) = refs
        c = lax.axis_index("c")
        s = lax.axis_index("s")
        wid = c * NS + s
        t0 = s * RT

        if with_deg:
            for i in range(16):
                zdb[i, :] = jnp.zeros((16,), jnp.float32)
            for i in range(C):
                ones[i, :] = jnp.ones((16,), jnp.float32)
        pltpu.sync_copy(zeros_hbm, acc.at[pl.ds(t0, RT), :])
        if with_deg:
            def zero_body(i, carry):
                pltpu.sync_copy(zdb, dacc.at[pl.ds(t0 + i * 16, 16), :])
                return carry
            lax.fori_loop(0, RT // 16, zero_body, 0)
        plsc.subcore_barrier()

        bufs = (rows0, rows1)
        gsems = (semg0, semg1)
        ssems = (sems0, sems1)

        def group_body(g, carry):
            pltpu.sync_copy(src_hbm.at[wid, g], srcv)
            pltpu.sync_copy(dst_hbm.at[wid, g], dstv)
            hg = [pltpu.async_copy(table_hbm.at[srcv.at[0]], rows0, semg0),
                  None]
            hs = [None, None]
            hd = None
            for j in range(GB):
                b = j % 2
                hg[b].wait()
                hs[b] = pltpu.async_copy(bufs[b], acc.at[dstv.at[j]],
                                         ssems[b], add=True)
                if with_deg:
                    if hd is not None:
                        hd.wait()
                    hd = pltpu.async_copy(ones, dacc.at[dstv.at[j]], semd,
                                          add=True)
                nb = 1 - b
                if j + 1 < GB:
                    if hs[nb] is not None:
                        hs[nb].wait()
                    hg[nb] = pltpu.async_copy(table_hbm.at[srcv.at[j + 1]],
                                              bufs[nb], gsems[nb])
            hs[0].wait()
            hs[1].wait()
            if with_deg:
                hd.wait()
            return carry
        lax.fori_loop(0, NG, group_body, 0)
        plsc.subcore_barrier()

        pltpu.sync_copy(acc.at[pl.ds(t0, RT), :], psum_hbm.at[c, pl.ds(t0, RT), :])
        if with_deg:
            pltpu.sync_copy(dacc.at[pl.ds(t0, RT), :],
                            pdeg_hbm.at[c, pl.ds(t0, RT), :])

    return agg


_agg_enc = _make_agg(IN_DIM, True)
_agg_dec = _make_agg(H_DIM, False)


def _enc_body(p_ref, d_ref, we_ref, wm_ref, z_ref):
    deg = jnp.clip(d_ref[0, :, :1] + d_ref[1, :, :1], 1.0, None)
    m = (p_ref[0] + p_ref[1]) / deg
    h = jnp.maximum(jnp.dot(m, we_ref[...], preferred_element_type=jnp.float32),
                    0.0)
    z_ref[...] = jnp.dot(h, wm_ref[...], preferred_element_type=jnp.float32)


def _dec_body(q_ref, d_ref, w1_ref, w2_ref, o_ref):
    deg = jnp.clip(d_ref[0, :, :1] + d_ref[1, :, :1], 1.0, None)
    m2 = (q_ref[0] + q_ref[1]) / deg
    h2 = jnp.maximum(jnp.dot(m2, w1_ref[...], preferred_element_type=jnp.float32),
                     0.0)
    o_ref[...] = jnp.dot(h2, w2_ref[...], preferred_element_type=jnp.float32)


def _dense(body, psum, pdeg, wa, wb, dout):
    din = psum.shape[-1]
    return pl.pallas_call(
        body,
        grid=(NBLK,),
        in_specs=[
            pl.BlockSpec((2, RB, din), lambda i: (0, i, 0)),
            pl.BlockSpec((2, RB, DEGW), lambda i: (0, i, 0)),
            pl.BlockSpec(wa.shape, lambda i: (0, 0)),
            pl.BlockSpec(wb.shape, lambda i: (0, 0)),
        ],
        out_specs=pl.BlockSpec((RB, dout), lambda i: (i, 0)),
        out_shape=jax.ShapeDtypeStruct((NPAD, dout), jnp.float32),
    )(psum, pdeg, wa, wb)


def kernel(x, edge_index, W_enc, W_mu, W_var, W_dec1, W_dec2):
    ei = edge_index.astype(jnp.int32)
    pad = EPAD - E
    apad = jnp.arange(pad, dtype=jnp.int32)
    src = jnp.concatenate([ei[0], apad % N]).reshape(NW, NG, GB, C)
    dst = jnp.concatenate([ei[1], N + apad % (NPAD - N)]).reshape(NW, NG, GB, C)
    z128 = jnp.zeros((RT, IN_DIM), jnp.float32)
    psum, pdeg = _agg_enc(x, src, dst, z128)
    z = _dense(_enc_body, psum, pdeg, W_enc, W_mu, H_DIM)
    (qsum,) = _agg_dec(z, src, dst, z128[:, :H_DIM])
    recon = _dense(_dec_body, qsum, pdeg, W_dec1, W_dec2, IN_DIM)
    return recon[:N]


# speedup vs baseline: 1.1001x; 1.1001x over previous
import functools

import jax
import jax.numpy as jnp
from jax import lax
from jax.experimental import pallas as pl
from jax.experimental.pallas import tpu as pltpu
from jax.experimental.pallas import tpu_sc as plsc

N = 10000
E = 320000
IN_DIM = 128
H_DIM = 64
HID = 512

NC = 2
NS = 16
NW = NC * NS
C = 128
GB = 8
NG = 10
T = NG * GB * C
EPAD = NW * T
NPAD = 10240
RT = NPAD // NS
RB = 640
NBLK = NPAD // RB
DEGW = 16


def _make_agg(d: int, with_deg: bool):
    out_type = [jax.ShapeDtypeStruct((NC, NPAD, d), jnp.float32)]
    scratch = [
        pltpu.VMEM((GB, C), jnp.int32),
        pltpu.VMEM((GB, C), jnp.int32),
        pltpu.VMEM((C, d), jnp.float32),
        pltpu.VMEM((C, d), jnp.float32),
        pltpu.VMEM_SHARED((NPAD, d), jnp.float32),
        pltpu.SemaphoreType.DMA,
        pltpu.SemaphoreType.DMA,
        pltpu.SemaphoreType.DMA,
        pltpu.SemaphoreType.DMA,
        pltpu.SemaphoreType.DMA,
    ]
    if with_deg:
        out_type.append(jax.ShapeDtypeStruct((NC, NPAD, DEGW), jnp.float32))
        scratch += [
            pltpu.VMEM((C, DEGW), jnp.float32),
            pltpu.VMEM((16, DEGW), jnp.float32),
            pltpu.VMEM_SHARED((NPAD, DEGW), jnp.float32),
        ]

    mesh = plsc.VectorSubcoreMesh(core_axis_name="c", subcore_axis_name="s")
    params = pltpu.CompilerParams(use_tc_tiling_on_sc=False)

    @functools.partial(pl.kernel, out_type=tuple(out_type), mesh=mesh,
                       scratch_types=scratch, compiler_params=params)
    def agg(table_hbm, src_hbm, dst_hbm, zeros_hbm, *refs):
        if with_deg:
            (psum_hbm, pdeg_hbm, srcv, dstv, rows0, rows1, acc,
             semg0, semg1, sems0, sems1, semd, ones, zdb, dacc) = refs
        else:
            (psum_hbm, srcv, dstv, rows0, rows1, acc,
             semg0, semg1, sems0, sems1, semd) = refs
        c = lax.axis_index("c")
        s = lax.axis_index("s")
        wid = c * NS + s
        t0 = s * RT

        if with_deg:
            for i in range(16):
                zdb[i, :] = jnp.zeros((16,), jnp.float32)
            for i in range(C):
                ones[i, :] = jnp.ones((16,), jnp.float32)
        pltpu.sync_copy(zeros_hbm, acc.at[pl.ds(t0, RT), :])
        if with_deg:
            def zero_body(i, carry):
                pltpu.sync_copy(zdb, dacc.at[pl.ds(t0 + i * 16, 16), :])
                return carry
            lax.fori_loop(0, RT // 16, zero_body, 0)
        plsc.subcore_barrier()

        bufs = (rows0, rows1)
        gsems = (semg0, semg1)
        ssems = (sems0, sems1)

        def group_body(g, carry):
            pltpu.sync_copy(src_hbm.at[wid, g], srcv)
            pltpu.sync_copy(dst_hbm.at[wid, g], dstv)
            pending = pltpu.async_copy(table_hbm.at[srcv.at[0]], rows0, semg0)
            hd = None
            for j in range(GB):
                b = j % 2
                if j + 1 < GB:
                    nxt = pltpu.async_copy(table_hbm.at[srcv.at[j + 1]],
                                           bufs[1 - b], gsems[1 - b])
                pending.wait()
                if j + 1 < GB:
                    pending = nxt
                pltpu.sync_copy(bufs[b], acc.at[dstv.at[j]], add=True)
                if with_deg:
                    if hd is not None:
                        hd.wait()
                    hd = pltpu.async_copy(ones, dacc.at[dstv.at[j]], semd,
                                          add=True)
            if with_deg:
                hd.wait()
            return carry
        lax.fori_loop(0, NG, group_body, 0)
        plsc.subcore_barrier()

        pltpu.sync_copy(acc.at[pl.ds(t0, RT), :], psum_hbm.at[c, pl.ds(t0, RT), :])
        if with_deg:
            pltpu.sync_copy(dacc.at[pl.ds(t0, RT), :],
                            pdeg_hbm.at[c, pl.ds(t0, RT), :])

    return agg


_agg_enc = _make_agg(IN_DIM, True)
_agg_dec = _make_agg(H_DIM, False)


def _enc_body(p_ref, d_ref, we_ref, wm_ref, z_ref):
    deg = jnp.clip(d_ref[0, :, :1] + d_ref[1, :, :1], 1.0, None)
    m = (p_ref[0] + p_ref[1]) / deg
    h = jnp.maximum(jnp.dot(m, we_ref[...], preferred_element_type=jnp.float32),
                    0.0)
    z_ref[...] = jnp.dot(h, wm_ref[...], preferred_element_type=jnp.float32)


def _dec_body(q_ref, d_ref, w1_ref, w2_ref, o_ref):
    deg = jnp.clip(d_ref[0, :, :1] + d_ref[1, :, :1], 1.0, None)
    m2 = (q_ref[0] + q_ref[1]) / deg
    h2 = jnp.maximum(jnp.dot(m2, w1_ref[...], preferred_element_type=jnp.float32),
                     0.0)
    o_ref[...] = jnp.dot(h2, w2_ref[...], preferred_element_type=jnp.float32)


def _dense(body, psum, pdeg, wa, wb, dout):
    din = psum.shape[-1]
    return pl.pallas_call(
        body,
        grid=(NBLK,),
        in_specs=[
            pl.BlockSpec((2, RB, din), lambda i: (0, i, 0)),
            pl.BlockSpec((2, RB, DEGW), lambda i: (0, i, 0)),
            pl.BlockSpec(wa.shape, lambda i: (0, 0)),
            pl.BlockSpec(wb.shape, lambda i: (0, 0)),
        ],
        out_specs=pl.BlockSpec((RB, dout), lambda i: (i, 0)),
        out_shape=jax.ShapeDtypeStruct((NPAD, dout), jnp.float32),
    )(psum, pdeg, wa, wb)


def kernel(x, edge_index, W_enc, W_mu, W_var, W_dec1, W_dec2):
    ei = edge_index.astype(jnp.int32)
    pad = EPAD - E
    apad = jnp.arange(pad, dtype=jnp.int32)
    src = jnp.concatenate([ei[0], apad % N]).reshape(NW, NG, GB, C)
    dst = jnp.concatenate([ei[1], N + apad % (NPAD - N)]).reshape(NW, NG, GB, C)
    z128 = jnp.zeros((RT, IN_DIM), jnp.float32)
    psum, pdeg = _agg_enc(x, src, dst, z128)
    z = _dense(_enc_body, psum, pdeg, W_enc, W_mu, H_DIM)
    (qsum,) = _agg_dec(z, src, dst, z128[:, :H_DIM])
    recon = _dense(_dec_body, qsum, pdeg, W_dec1, W_dec2, IN_DIM)
    return recon[:N]


# speedup vs baseline: 1.1097x; 1.0086x over previous
import functools

import jax
import jax.numpy as jnp
from jax import lax
from jax.experimental import pallas as pl
from jax.experimental.pallas import tpu as pltpu
from jax.experimental.pallas import tpu_sc as plsc

N = 10000
E = 320000
IN_DIM = 128
H_DIM = 64
HID = 512

NC = 2
NS = 16
NW = NC * NS
C = 128
GB = 8
NG = 10
T = NG * GB * C
EPAD = NW * T
NPAD = 10240
RT = NPAD // NS
RB = 640
NBLK = NPAD // RB
DEGW = 16


def _make_agg(d: int, with_deg: bool):
    nbuf = 2 if with_deg else 4
    out_type = [jax.ShapeDtypeStruct((NC, NPAD, d), jnp.float32)]
    scratch = (
        [pltpu.VMEM((GB, C), jnp.int32)] * 2
        + [pltpu.VMEM((C, d), jnp.float32)] * nbuf
        + [pltpu.VMEM_SHARED((NPAD, d), jnp.float32)]
        + [pltpu.SemaphoreType.DMA] * (2 * nbuf + 1)
    )
    if with_deg:
        out_type.append(jax.ShapeDtypeStruct((NC, NPAD, DEGW), jnp.float32))
        scratch += [
            pltpu.VMEM((C, DEGW), jnp.float32),
            pltpu.VMEM((16, DEGW), jnp.float32),
            pltpu.VMEM_SHARED((NPAD, DEGW), jnp.float32),
        ]

    mesh = plsc.VectorSubcoreMesh(core_axis_name="c", subcore_axis_name="s")
    params = pltpu.CompilerParams(use_tc_tiling_on_sc=False)

    @functools.partial(pl.kernel, out_type=tuple(out_type), mesh=mesh,
                       scratch_types=scratch, compiler_params=params)
    def agg(table_hbm, src_hbm, dst_hbm, zeros_hbm, *refs):
        no = 2 if with_deg else 1
        psum_hbm = refs[0]
        srcv, dstv = refs[no], refs[no + 1]
        bufs = refs[no + 2:no + 2 + nbuf]
        acc = refs[no + 2 + nbuf]
        gsems = refs[no + 3 + nbuf:no + 3 + 2 * nbuf]
        ssems = refs[no + 3 + 2 * nbuf:no + 3 + 3 * nbuf]
        semd = refs[no + 3 + 3 * nbuf]
        if with_deg:
            pdeg_hbm = refs[1]
            ones, zdb, dacc = refs[no + 4 + 3 * nbuf:]
        c = lax.axis_index("c")
        s = lax.axis_index("s")
        wid = c * NS + s
        t0 = s * RT

        if with_deg:
            for i in range(16):
                zdb[i, :] = jnp.zeros((16,), jnp.float32)
            for i in range(C):
                ones[i, :] = jnp.ones((16,), jnp.float32)
        pltpu.sync_copy(zeros_hbm, acc.at[pl.ds(t0, RT), :])
        if with_deg:
            def zero_body(i, carry):
                pltpu.sync_copy(zdb, dacc.at[pl.ds(t0 + i * 16, 16), :])
                return carry
            lax.fori_loop(0, RT // 16, zero_body, 0)
        plsc.subcore_barrier()

        def group_body2(g, carry):
            pltpu.sync_copy(src_hbm.at[wid, g], srcv)
            pltpu.sync_copy(dst_hbm.at[wid, g], dstv)
            pending = pltpu.async_copy(table_hbm.at[srcv.at[0]], bufs[0],
                                       gsems[0])
            hd = None
            for j in range(GB):
                b = j % 2
                if j + 1 < GB:
                    nxt = pltpu.async_copy(table_hbm.at[srcv.at[j + 1]],
                                           bufs[1 - b], gsems[1 - b])
                pending.wait()
                if j + 1 < GB:
                    pending = nxt
                pltpu.sync_copy(bufs[b], acc.at[dstv.at[j]], add=True)
                if with_deg:
                    if hd is not None:
                        hd.wait()
                    hd = pltpu.async_copy(ones, dacc.at[dstv.at[j]], semd,
                                          add=True)
            if with_deg:
                hd.wait()
            return carry

        def group_body4(g, carry):
            pltpu.sync_copy(src_hbm.at[wid, g], srcv)
            pltpu.sync_copy(dst_hbm.at[wid, g], dstv)
            hg = [None] * 4
            hs = [None] * 4
            for b in range(2):
                hg[b] = pltpu.async_copy(table_hbm.at[srcv.at[b]], bufs[b],
                                         gsems[b])
            for j in range(GB):
                b = j % 4
                hg[b].wait()
                hs[b] = pltpu.async_copy(bufs[b], acc.at[dstv.at[j]],
                                         ssems[b], add=True)
                if j + 2 < GB:
                    b2 = (j + 2) % 4
                    if hs[b2] is not None:
                        hs[b2].wait()
                    hg[b2] = pltpu.async_copy(table_hbm.at[srcv.at[j + 2]],
                                              bufs[b2], gsems[b2])
            for b in range(4):
                hs[b].wait()
            return carry

        lax.fori_loop(0, NG, group_body2 if with_deg else group_body4, 0)
        plsc.subcore_barrier()

        pltpu.sync_copy(acc.at[pl.ds(t0, RT), :], psum_hbm.at[c, pl.ds(t0, RT), :])
        if with_deg:
            pltpu.sync_copy(dacc.at[pl.ds(t0, RT), :],
                            pdeg_hbm.at[c, pl.ds(t0, RT), :])

    return agg


_agg_enc = _make_agg(IN_DIM, True)
_agg_dec = _make_agg(H_DIM, False)


def _enc_body(p_ref, d_ref, we_ref, wm_ref, z_ref):
    deg = jnp.clip(d_ref[0, :, :1] + d_ref[1, :, :1], 1.0, None)
    m = (p_ref[0] + p_ref[1]) / deg
    h = jnp.maximum(jnp.dot(m, we_ref[...], preferred_element_type=jnp.float32),
                    0.0)
    z_ref[...] = jnp.dot(h, wm_ref[...], preferred_element_type=jnp.float32)


def _dec_body(q_ref, d_ref, w1_ref, w2_ref, o_ref):
    deg = jnp.clip(d_ref[0, :, :1] + d_ref[1, :, :1], 1.0, None)
    m2 = (q_ref[0] + q_ref[1]) / deg
    h2 = jnp.maximum(jnp.dot(m2, w1_ref[...], preferred_element_type=jnp.float32),
                     0.0)
    o_ref[...] = jnp.dot(h2, w2_ref[...], preferred_element_type=jnp.float32)


def _dense(body, psum, pdeg, wa, wb, dout):
    din = psum.shape[-1]
    return pl.pallas_call(
        body,
        grid=(NBLK,),
        in_specs=[
            pl.BlockSpec((2, RB, din), lambda i: (0, i, 0)),
            pl.BlockSpec((2, RB, DEGW), lambda i: (0, i, 0)),
            pl.BlockSpec(wa.shape, lambda i: (0, 0)),
            pl.BlockSpec(wb.shape, lambda i: (0, 0)),
        ],
        out_specs=pl.BlockSpec((RB, dout), lambda i: (i, 0)),
        out_shape=jax.ShapeDtypeStruct((NPAD, dout), jnp.float32),
    )(psum, pdeg, wa, wb)


def kernel(x, edge_index, W_enc, W_mu, W_var, W_dec1, W_dec2):
    ei = edge_index.astype(jnp.int32)
    pad = EPAD - E
    apad = jnp.arange(pad, dtype=jnp.int32)
    src = jnp.concatenate([ei[0], apad % N]).reshape(NW, NG, GB, C)
    dst = jnp.concatenate([ei[1], N + apad % (NPAD - N)]).reshape(NW, NG, GB, C)
    z128 = jnp.zeros((RT, IN_DIM), jnp.float32)
    psum, pdeg = _agg_enc(x, src, dst, z128)
    z = _dense(_enc_body, psum, pdeg, W_enc, W_mu, H_DIM)
    (qsum,) = _agg_dec(z, src, dst, z128[:, :H_DIM])
    recon = _dense(_dec_body, qsum, pdeg, W_dec1, W_dec2, IN_DIM)
    return recon[:N]
